# Initial kernel scaffold; baseline (speedup 1.0000x reference)
import jax
import jax.numpy as jnp
from jax.experimental import pallas as pl


def kernel(xyz_fine, xyz_coarse, feats_fine, feats_coarse, W1, b1, g1, be1, W2, b2, g2, be2):
    raise NotImplementedError("write your pallas kernel here")


# speedup vs baseline: 19.8789x; 19.8789x over previous
import jax
import jax.numpy as jnp
from jax.experimental import pallas as pl

_HI = jax.lax.Precision.HIGHEST


def _stage_a_kernel(xyzf_ref, xyzct_ref, ff_ref, fc_ref, w1_ref, b1_ref,
                    y1_ref, s_ref, ss_ref):
    b = pl.program_id(0)
    n = pl.program_id(1)

    x = xyzf_ref[0]
    ct = xyzct_ref[0]
    d = -2.0 * jax.lax.dot_general(x, ct, (((1,), (0,)), ((), ())))
    d = d + jnp.sum(x * x, axis=1, keepdims=True)
    d = d + jnp.sum(ct * ct, axis=0, keepdims=True)

    m = d.shape[1]
    col = jax.lax.broadcasted_iota(jnp.int32, d.shape, 1)

    dwork = d
    iks = []
    mks = []
    for k in range(3):
        mk = jnp.min(dwork, axis=1, keepdims=True)
        ik = jnp.min(jnp.where(dwork == mk, col, m), axis=1, keepdims=True)
        mks.append(mk)
        iks.append(ik)
        if k < 2:
            dwork = jnp.where(col == ik, jnp.float32(jnp.inf), dwork)

    d3 = jnp.concatenate(mks, axis=1)
    d3 = jnp.maximum(d3, jnp.float32(1e-10))
    w = 1.0 / d3
    w = w / jnp.sum(w, axis=1, keepdims=True)

    wmat = jnp.where(col == iks[0], w[:, 0:1], 0.0)
    wmat = wmat + jnp.where(col == iks[1], w[:, 1:2], 0.0)
    wmat = wmat + jnp.where(col == iks[2], w[:, 2:3], 0.0)
    interp = jax.lax.dot_general(wmat, fc_ref[0], (((1,), (0,)), ((), ())),
                                 precision=_HI)

    ff = ff_ref[0]
    w1 = w1_ref[...]
    c2 = interp.shape[1]
    y = jax.lax.dot_general(interp, w1[:, :c2], (((1,), (1,)), ((), ())),
                            precision=_HI)
    y = y + jax.lax.dot_general(ff, w1[:, c2:], (((1,), (1,)), ((), ())),
                                precision=_HI)
    y = y + b1_ref[...]
    y1_ref[0] = y

    @pl.when((b == 0) & (n == 0))
    def _init():
        s_ref[...] = jnp.zeros_like(s_ref)
        ss_ref[...] = jnp.zeros_like(ss_ref)

    s_ref[...] += jnp.sum(y, axis=0, keepdims=True)
    ss_ref[...] += jnp.sum(y * y, axis=0, keepdims=True)


def _stage_b_kernel(y_ref, sc_ref, sh_ref, w2_ref, b2_ref,
                    o_ref, s_ref, ss_ref):
    i = pl.program_id(0)
    z = jnp.maximum(y_ref[...] * sc_ref[...] + sh_ref[...], 0.0)
    y = jax.lax.dot_general(z, w2_ref[...], (((1,), (1,)), ((), ())),
                            precision=_HI) + b2_ref[...]
    o_ref[...] = y

    @pl.when(i == 0)
    def _init():
        s_ref[...] = jnp.zeros_like(s_ref)
        ss_ref[...] = jnp.zeros_like(ss_ref)

    s_ref[...] += jnp.sum(y, axis=0, keepdims=True)
    ss_ref[...] += jnp.sum(y * y, axis=0, keepdims=True)


def _stage_c_kernel(y_ref, sc_ref, sh_ref, o_ref):
    o_ref[...] = jnp.maximum(y_ref[...] * sc_ref[...] + sh_ref[...], 0.0)


def kernel(xyz_fine, xyz_coarse, feats_fine, feats_coarse,
           W1, b1, g1, be1, W2, b2, g2, be2):
    B, N, _ = xyz_fine.shape
    M = xyz_coarse.shape[1]
    C1 = feats_fine.shape[2]
    C2 = feats_coarse.shape[2]
    H = W1.shape[0]
    eps = jnp.float32(1e-5)
    cnt = jnp.float32(B * N)

    blk = 512
    xyz_coarse_t = jnp.transpose(xyz_coarse, (0, 2, 1))
    b1r = b1.reshape(1, H)

    y1, s1, ss1 = pl.pallas_call(
        _stage_a_kernel,
        grid=(B, N // blk),
        in_specs=[
            pl.BlockSpec((1, blk, 3), lambda b, n: (b, n, 0)),
            pl.BlockSpec((1, 3, M), lambda b, n: (b, 0, 0)),
            pl.BlockSpec((1, blk, C1), lambda b, n: (b, n, 0)),
            pl.BlockSpec((1, M, C2), lambda b, n: (b, 0, 0)),
            pl.BlockSpec((H, C1 + C2), lambda b, n: (0, 0)),
            pl.BlockSpec((1, H), lambda b, n: (0, 0)),
        ],
        out_specs=[
            pl.BlockSpec((1, blk, H), lambda b, n: (b, n, 0)),
            pl.BlockSpec((1, H), lambda b, n: (0, 0)),
            pl.BlockSpec((1, H), lambda b, n: (0, 0)),
        ],
        out_shape=[
            jax.ShapeDtypeStruct((B, N, H), jnp.float32),
            jax.ShapeDtypeStruct((1, H), jnp.float32),
            jax.ShapeDtypeStruct((1, H), jnp.float32),
        ],
    )(xyz_fine, xyz_coarse_t, feats_fine, feats_coarse, W1, b1r)

    mean1 = s1 / cnt
    var1 = ss1 / cnt - mean1 * mean1
    sc1 = g1.reshape(1, H) / jnp.sqrt(var1 + eps)
    sh1 = be1.reshape(1, H) - mean1 * sc1

    y1f = y1.reshape(B * N, H)
    blk2 = 2048
    y2, s2, ss2 = pl.pallas_call(
        _stage_b_kernel,
        grid=(B * N // blk2,),
        in_specs=[
            pl.BlockSpec((blk2, H), lambda i: (i, 0)),
            pl.BlockSpec((1, H), lambda i: (0, 0)),
            pl.BlockSpec((1, H), lambda i: (0, 0)),
            pl.BlockSpec((H, H), lambda i: (0, 0)),
            pl.BlockSpec((1, H), lambda i: (0, 0)),
        ],
        out_specs=[
            pl.BlockSpec((blk2, H), lambda i: (i, 0)),
            pl.BlockSpec((1, H), lambda i: (0, 0)),
            pl.BlockSpec((1, H), lambda i: (0, 0)),
        ],
        out_shape=[
            jax.ShapeDtypeStruct((B * N, H), jnp.float32),
            jax.ShapeDtypeStruct((1, H), jnp.float32),
            jax.ShapeDtypeStruct((1, H), jnp.float32),
        ],
    )(y1f, sc1, sh1, W2, b2.reshape(1, H))

    mean2 = s2 / cnt
    var2 = ss2 / cnt - mean2 * mean2
    sc2 = g2.reshape(1, H) / jnp.sqrt(var2 + eps)
    sh2 = be2.reshape(1, H) - mean2 * sc2

    out = pl.pallas_call(
        _stage_c_kernel,
        grid=(B * N // blk2,),
        in_specs=[
            pl.BlockSpec((blk2, H), lambda i: (i, 0)),
            pl.BlockSpec((1, H), lambda i: (0, 0)),
            pl.BlockSpec((1, H), lambda i: (0, 0)),
        ],
        out_specs=pl.BlockSpec((blk2, H), lambda i: (i, 0)),
        out_shape=jax.ShapeDtypeStruct((B * N, H), jnp.float32),
    )(y2, sc2, sh2)

    return out.reshape(B, N, H)


# speedup vs baseline: 33.7879x; 1.6997x over previous
import functools

import jax
import jax.numpy as jnp
from jax import lax
from jax.experimental import pallas as pl
from jax.experimental.pallas import tpu as pltpu
from jax.experimental.pallas import tpu_sc as plsc

_HI = jax.lax.Precision.HIGHEST

_NC = 2
_NS = 16
_NW = _NC * _NS


def _knn_kernel(xyzf_ref, xyzct_ref, idx_ref, w_ref):
    b = pl.program_id(0)

    x = xyzf_ref[0]
    ct = xyzct_ref[0]
    d = -2.0 * jax.lax.dot_general(x, ct, (((1,), (0,)), ((), ())))
    d = d + jnp.sum(x * x, axis=1, keepdims=True)
    d = d + jnp.sum(ct * ct, axis=0, keepdims=True)

    m = d.shape[1]
    col = jax.lax.broadcasted_iota(jnp.int32, d.shape, 1)

    dwork = d
    iks = []
    mks = []
    for k in range(3):
        mk = jnp.min(dwork, axis=1, keepdims=True)
        ik = jnp.min(jnp.where(dwork == mk, col, m), axis=1, keepdims=True)
        mks.append(mk)
        iks.append(ik)
        if k < 2:
            dwork = jnp.where(col == ik, jnp.float32(jnp.inf), dwork)

    d3 = jnp.concatenate(mks, axis=1)
    d3 = jnp.maximum(d3, jnp.float32(1e-10))
    w = 1.0 / d3
    w = w / jnp.sum(w, axis=1, keepdims=True)

    ig = jnp.concatenate(iks, axis=1) + b * m
    idx_ref[...] = jnp.transpose(ig, (1, 0)).reshape(3, 1, -1)
    w_ref[0] = w


def _sc_gather_body(idx_hbm, fc_hbm, g0_hbm, g1_hbm, g2_hbm,
                    i0, i1, i2, r0, r1, r2, sem):
    p_per_w = idx_hbm.shape[2] // _NW
    chunk = i0.shape[0]
    wid = lax.axis_index("s") * _NC + lax.axis_index("c")
    base = wid * p_per_w

    def chunk_body(ci, carry):
        cbase = base + ci * chunk
        pltpu.sync_copy(idx_hbm.at[0, 0, pl.ds(cbase, chunk)], i0)
        pltpu.sync_copy(idx_hbm.at[1, 0, pl.ds(cbase, chunk)], i1)
        pltpu.sync_copy(idx_hbm.at[2, 0, pl.ds(cbase, chunk)], i2)
        c0 = pltpu.async_copy(fc_hbm.at[i0], r0, sem)
        c1 = pltpu.async_copy(fc_hbm.at[i1], r1, sem)
        c2 = pltpu.async_copy(fc_hbm.at[i2], r2, sem)
        c0.wait()
        c1.wait()
        c2.wait()
        pltpu.sync_copy(r0, g0_hbm.at[pl.ds(cbase, chunk)])
        pltpu.sync_copy(r1, g1_hbm.at[pl.ds(cbase, chunk)])
        pltpu.sync_copy(r2, g2_hbm.at[pl.ds(cbase, chunk)])
        return carry

    lax.fori_loop(0, p_per_w // chunk, chunk_body, 0)


def _mlp1_kernel(g0_ref, g1_ref, g2_ref, w_ref, ff_ref, w1_ref, b1_ref,
                 y1_ref, s_ref, ss_ref):
    i = pl.program_id(0)
    w = w_ref[...]
    c2 = w1_ref.shape[1] - ff_ref.shape[1]
    it = (w[:, 0:1] * g0_ref[:, :c2] + w[:, 1:2] * g1_ref[:, :c2]
          + w[:, 2:3] * g2_ref[:, :c2])
    ff = ff_ref[...]
    w1 = w1_ref[...]
    y = jax.lax.dot_general(it, w1[:, :c2], (((1,), (1,)), ((), ())),
                            precision=_HI)
    y = y + jax.lax.dot_general(ff, w1[:, c2:], (((1,), (1,)), ((), ())),
                                precision=_HI)
    y = y + b1_ref[...]
    y1_ref[...] = y

    @pl.when(i == 0)
    def _init():
        s_ref[...] = jnp.zeros_like(s_ref)
        ss_ref[...] = jnp.zeros_like(ss_ref)

    s_ref[...] += jnp.sum(y, axis=0, keepdims=True)
    ss_ref[...] += jnp.sum(y * y, axis=0, keepdims=True)


def _mlp2_kernel(y_ref, sc_ref, sh_ref, w2_ref, b2_ref,
                 o_ref, s_ref, ss_ref):
    i = pl.program_id(0)
    z = jnp.maximum(y_ref[...] * sc_ref[...] + sh_ref[...], 0.0)
    y = jax.lax.dot_general(z, w2_ref[...], (((1,), (1,)), ((), ())),
                            precision=_HI) + b2_ref[...]
    o_ref[...] = y

    @pl.when(i == 0)
    def _init():
        s_ref[...] = jnp.zeros_like(s_ref)
        ss_ref[...] = jnp.zeros_like(ss_ref)

    s_ref[...] += jnp.sum(y, axis=0, keepdims=True)
    ss_ref[...] += jnp.sum(y * y, axis=0, keepdims=True)


def _bn_out_kernel(y_ref, sc_ref, sh_ref, o_ref):
    o_ref[...] = jnp.maximum(y_ref[...] * sc_ref[...] + sh_ref[...], 0.0)


def kernel(xyz_fine, xyz_coarse, feats_fine, feats_coarse,
           W1, b1, g1, be1, W2, b2, g2, be2):
    B, N, _ = xyz_fine.shape
    M = xyz_coarse.shape[1]
    C1 = feats_fine.shape[2]
    C2 = feats_coarse.shape[2]
    H = W1.shape[0]
    P = B * N
    eps = jnp.float32(1e-5)
    cnt = jnp.float32(P)

    blk = 512
    nb = N // blk
    xyz_coarse_t = jnp.transpose(xyz_coarse, (0, 2, 1))

    idx_t, w_t = pl.pallas_call(
        _knn_kernel,
        grid=(B, nb),
        in_specs=[
            pl.BlockSpec((1, blk, 3), lambda b, n: (b, n, 0)),
            pl.BlockSpec((1, 3, M), lambda b, n: (b, 0, 0)),
        ],
        out_specs=[
            pl.BlockSpec((3, 1, blk), lambda b, n: (0, 0, b * (N // 512) + n)),
            pl.BlockSpec((1, blk, 3), lambda b, n: (b, n, 0)),
        ],
        out_shape=[
            jax.ShapeDtypeStruct((3, 1, P), jnp.int32),
            jax.ShapeDtypeStruct((B, N, 3), jnp.float32),
        ],
    )(xyz_fine, xyz_coarse_t)

    fc_flat = feats_coarse.reshape(B * M, C2)
    fc_pad = jnp.pad(fc_flat, ((0, 0), (0, 128 - C2)))
    chunk = 256
    sc_gather = functools.partial(
        pl.kernel,
        mesh=plsc.VectorSubcoreMesh(core_axis_name="c", subcore_axis_name="s"),
        out_type=[
            jax.ShapeDtypeStruct((P, 128), jnp.float32),
            jax.ShapeDtypeStruct((P, 128), jnp.float32),
            jax.ShapeDtypeStruct((P, 128), jnp.float32),
        ],
        scratch_types=[
            pltpu.VMEM((chunk,), jnp.int32),
            pltpu.VMEM((chunk,), jnp.int32),
            pltpu.VMEM((chunk,), jnp.int32),
            pltpu.VMEM((chunk, 128), jnp.float32),
            pltpu.VMEM((chunk, 128), jnp.float32),
            pltpu.VMEM((chunk, 128), jnp.float32),
            pltpu.SemaphoreType.DMA,
        ],
    )(_sc_gather_body)
    ga0, ga1, ga2 = sc_gather(idx_t, fc_pad)

    ff_flat = feats_fine.reshape(P, C1)
    w_flat = w_t.reshape(P, 3)
    blk2 = 2048
    y1, s1, ss1 = pl.pallas_call(
        _mlp1_kernel,
        grid=(P // blk2,),
        in_specs=[
            pl.BlockSpec((blk2, 128), lambda i: (i, 0)),
            pl.BlockSpec((blk2, 128), lambda i: (i, 0)),
            pl.BlockSpec((blk2, 128), lambda i: (i, 0)),
            pl.BlockSpec((blk2, 3), lambda i: (i, 0)),
            pl.BlockSpec((blk2, C1), lambda i: (i, 0)),
            pl.BlockSpec((H, C1 + C2), lambda i: (0, 0)),
            pl.BlockSpec((1, H), lambda i: (0, 0)),
        ],
        out_specs=[
            pl.BlockSpec((blk2, H), lambda i: (i, 0)),
            pl.BlockSpec((1, H), lambda i: (0, 0)),
            pl.BlockSpec((1, H), lambda i: (0, 0)),
        ],
        out_shape=[
            jax.ShapeDtypeStruct((P, H), jnp.float32),
            jax.ShapeDtypeStruct((1, H), jnp.float32),
            jax.ShapeDtypeStruct((1, H), jnp.float32),
        ],
    )(ga0, ga1, ga2, w_flat, ff_flat, W1, b1.reshape(1, H))

    mean1 = s1 / cnt
    var1 = ss1 / cnt - mean1 * mean1
    sc1 = g1.reshape(1, H) / jnp.sqrt(var1 + eps)
    sh1 = be1.reshape(1, H) - mean1 * sc1

    y2, s2, ss2 = pl.pallas_call(
        _mlp2_kernel,
        grid=(P // blk2,),
        in_specs=[
            pl.BlockSpec((blk2, H), lambda i: (i, 0)),
            pl.BlockSpec((1, H), lambda i: (0, 0)),
            pl.BlockSpec((1, H), lambda i: (0, 0)),
            pl.BlockSpec((H, H), lambda i: (0, 0)),
            pl.BlockSpec((1, H), lambda i: (0, 0)),
        ],
        out_specs=[
            pl.BlockSpec((blk2, H), lambda i: (i, 0)),
            pl.BlockSpec((1, H), lambda i: (0, 0)),
            pl.BlockSpec((1, H), lambda i: (0, 0)),
        ],
        out_shape=[
            jax.ShapeDtypeStruct((P, H), jnp.float32),
            jax.ShapeDtypeStruct((1, H), jnp.float32),
            jax.ShapeDtypeStruct((1, H), jnp.float32),
        ],
    )(y1, sc1, sh1, W2, b2.reshape(1, H))

    mean2 = s2 / cnt
    var2 = ss2 / cnt - mean2 * mean2
    sc2 = g2.reshape(1, H) / jnp.sqrt(var2 + eps)
    sh2 = be2.reshape(1, H) - mean2 * sc2

    out = pl.pallas_call(
        _bn_out_kernel,
        grid=(P // blk2,),
        in_specs=[
            pl.BlockSpec((blk2, H), lambda i: (i, 0)),
            pl.BlockSpec((1, H), lambda i: (0, 0)),
            pl.BlockSpec((1, H), lambda i: (0, 0)),
        ],
        out_specs=pl.BlockSpec((blk2, H), lambda i: (i, 0)),
        out_shape=jax.ShapeDtypeStruct((P, H), jnp.float32),
    )(y2, sc2, sh2)

    return out.reshape(B, N, H)


# speedup vs baseline: 43.1823x; 1.2780x over previous
import functools

import jax
import jax.numpy as jnp
from jax import lax
from jax.experimental import pallas as pl
from jax.experimental.pallas import tpu as pltpu
from jax.experimental.pallas import tpu_sc as plsc

_HI = jax.lax.Precision.HIGHEST

_NC = 2
_NS = 16
_NW = _NC * _NS


def _knn_kernel(xyzf_ref, xyzct_ref, idx_ref, w_ref):
    b = pl.program_id(0)

    x = xyzf_ref[0]
    ct = xyzct_ref[0]
    d = -2.0 * jax.lax.dot_general(x, ct, (((1,), (0,)), ((), ())))
    d = d + jnp.sum(x * x, axis=1, keepdims=True)
    d = d + jnp.sum(ct * ct, axis=0, keepdims=True)

    m = d.shape[1]
    coli = jax.lax.broadcasted_iota(jnp.int32, d.shape, 1)
    dc = jnp.maximum(d, jnp.float32(1e-10))
    key = jax.lax.bitcast_convert_type(dc, jnp.int32)
    key = jnp.bitwise_or(jnp.bitwise_and(key, jnp.int32(-2048)), coli)
    kw = jax.lax.bitcast_convert_type(key, jnp.float32)

    mks = []
    for k in range(3):
        mk = jnp.min(kw, axis=1, keepdims=True)
        mks.append(mk)
        if k < 2:
            kw = jnp.where(kw == mk, jnp.float32(jnp.inf), kw)

    k3 = jax.lax.bitcast_convert_type(jnp.concatenate(mks, axis=1),
                                      jnp.int32)
    ig = jnp.bitwise_and(k3, jnp.int32(2047)) + b * m
    d3 = jax.lax.bitcast_convert_type(
        jnp.bitwise_and(k3, jnp.int32(-2048)), jnp.float32)
    d3 = jnp.maximum(d3, jnp.float32(1e-10))
    w = 1.0 / d3
    w = w / jnp.sum(w, axis=1, keepdims=True)

    idx_ref[...] = jnp.transpose(ig, (1, 0)).reshape(3, 1, -1)
    w_ref[0] = w


def _sc_gather_body(idx_hbm, fc_hbm, g0_hbm, g1_hbm, g2_hbm,
                    i0, i1, i2, r0, r1, r2, sem):
    p_per_w = idx_hbm.shape[2] // _NW
    chunk = i0.shape[0]
    wid = lax.axis_index("s") * _NC + lax.axis_index("c")
    base = wid * p_per_w

    def chunk_body(ci, carry):
        cbase = base + ci * chunk
        pltpu.sync_copy(idx_hbm.at[0, 0, pl.ds(cbase, chunk)], i0)
        pltpu.sync_copy(idx_hbm.at[1, 0, pl.ds(cbase, chunk)], i1)
        pltpu.sync_copy(idx_hbm.at[2, 0, pl.ds(cbase, chunk)], i2)
        c0 = pltpu.async_copy(fc_hbm.at[i0], r0, sem)
        c1 = pltpu.async_copy(fc_hbm.at[i1], r1, sem)
        c2 = pltpu.async_copy(fc_hbm.at[i2], r2, sem)
        c0.wait()
        c1.wait()
        c2.wait()
        pltpu.sync_copy(r0, g0_hbm.at[pl.ds(cbase, chunk)])
        pltpu.sync_copy(r1, g1_hbm.at[pl.ds(cbase, chunk)])
        pltpu.sync_copy(r2, g2_hbm.at[pl.ds(cbase, chunk)])
        return carry

    lax.fori_loop(0, p_per_w // chunk, chunk_body, 0)


def _mlp1_kernel(g0_ref, g1_ref, g2_ref, w_ref, ff_ref, w1_ref, b1_ref,
                 y1_ref, s_ref, ss_ref):
    i = pl.program_id(0)
    w = w_ref[...]
    c2 = w1_ref.shape[1] - ff_ref.shape[1]
    it = (w[:, 0:1] * g0_ref[:, :c2] + w[:, 1:2] * g1_ref[:, :c2]
          + w[:, 2:3] * g2_ref[:, :c2])
    ff = ff_ref[...]
    w1 = w1_ref[...]
    y = jax.lax.dot_general(it, w1[:, :c2], (((1,), (1,)), ((), ())),
                            precision=_HI)
    y = y + jax.lax.dot_general(ff, w1[:, c2:], (((1,), (1,)), ((), ())),
                                precision=_HI)
    y = y + b1_ref[...]
    y1_ref[...] = y

    @pl.when(i == 0)
    def _init():
        s_ref[...] = jnp.zeros_like(s_ref)
        ss_ref[...] = jnp.zeros_like(ss_ref)

    s_ref[...] += jnp.sum(y, axis=0, keepdims=True)
    ss_ref[...] += jnp.sum(y * y, axis=0, keepdims=True)


def _mlp2_kernel(y_ref, sc_ref, sh_ref, w2_ref, b2_ref,
                 o_ref, s_ref, ss_ref):
    i = pl.program_id(0)
    z = jnp.maximum(y_ref[...] * sc_ref[...] + sh_ref[...], 0.0)
    y = jax.lax.dot_general(z, w2_ref[...], (((1,), (1,)), ((), ())),
                            precision=_HI) + b2_ref[...]
    o_ref[...] = y

    @pl.when(i == 0)
    def _init():
        s_ref[...] = jnp.zeros_like(s_ref)
        ss_ref[...] = jnp.zeros_like(ss_ref)

    s_ref[...] += jnp.sum(y, axis=0, keepdims=True)
    ss_ref[...] += jnp.sum(y * y, axis=0, keepdims=True)


def _bn_out_kernel(y_ref, sc_ref, sh_ref, o_ref):
    o_ref[...] = jnp.maximum(y_ref[...] * sc_ref[...] + sh_ref[...], 0.0)


def kernel(xyz_fine, xyz_coarse, feats_fine, feats_coarse,
           W1, b1, g1, be1, W2, b2, g2, be2):
    B, N, _ = xyz_fine.shape
    M = xyz_coarse.shape[1]
    C1 = feats_fine.shape[2]
    C2 = feats_coarse.shape[2]
    H = W1.shape[0]
    P = B * N
    eps = jnp.float32(1e-5)
    cnt = jnp.float32(P)

    blk = 512
    nb = N // blk
    xyz_coarse_t = jnp.transpose(xyz_coarse, (0, 2, 1))

    idx_t, w_t = pl.pallas_call(
        _knn_kernel,
        grid=(B, nb),
        in_specs=[
            pl.BlockSpec((1, blk, 3), lambda b, n: (b, n, 0)),
            pl.BlockSpec((1, 3, M), lambda b, n: (b, 0, 0)),
        ],
        out_specs=[
            pl.BlockSpec((3, 1, blk), lambda b, n: (0, 0, b * (N // 512) + n)),
            pl.BlockSpec((1, blk, 3), lambda b, n: (b, n, 0)),
        ],
        out_shape=[
            jax.ShapeDtypeStruct((3, 1, P), jnp.int32),
            jax.ShapeDtypeStruct((B, N, 3), jnp.float32),
        ],
    )(xyz_fine, xyz_coarse_t)

    fc_flat = feats_coarse.reshape(B * M, C2)
    fc_pad = jnp.pad(fc_flat, ((0, 0), (0, 128 - C2)))
    chunk = 256
    sc_gather = functools.partial(
        pl.kernel,
        mesh=plsc.VectorSubcoreMesh(core_axis_name="c", subcore_axis_name="s"),
        out_type=[
            jax.ShapeDtypeStruct((P, 128), jnp.float32),
            jax.ShapeDtypeStruct((P, 128), jnp.float32),
            jax.ShapeDtypeStruct((P, 128), jnp.float32),
        ],
        scratch_types=[
            pltpu.VMEM((chunk,), jnp.int32),
            pltpu.VMEM((chunk,), jnp.int32),
            pltpu.VMEM((chunk,), jnp.int32),
            pltpu.VMEM((chunk, 128), jnp.float32),
            pltpu.VMEM((chunk, 128), jnp.float32),
            pltpu.VMEM((chunk, 128), jnp.float32),
            pltpu.SemaphoreType.DMA,
        ],
    )(_sc_gather_body)
    ga0, ga1, ga2 = sc_gather(idx_t, fc_pad)

    ff_flat = feats_fine.reshape(P, C1)
    w_flat = w_t.reshape(P, 3)
    blk2 = 2048
    y1, s1, ss1 = pl.pallas_call(
        _mlp1_kernel,
        grid=(P // blk2,),
        in_specs=[
            pl.BlockSpec((blk2, 128), lambda i: (i, 0)),
            pl.BlockSpec((blk2, 128), lambda i: (i, 0)),
            pl.BlockSpec((blk2, 128), lambda i: (i, 0)),
            pl.BlockSpec((blk2, 3), lambda i: (i, 0)),
            pl.BlockSpec((blk2, C1), lambda i: (i, 0)),
            pl.BlockSpec((H, C1 + C2), lambda i: (0, 0)),
            pl.BlockSpec((1, H), lambda i: (0, 0)),
        ],
        out_specs=[
            pl.BlockSpec((blk2, H), lambda i: (i, 0)),
            pl.BlockSpec((1, H), lambda i: (0, 0)),
            pl.BlockSpec((1, H), lambda i: (0, 0)),
        ],
        out_shape=[
            jax.ShapeDtypeStruct((P, H), jnp.float32),
            jax.ShapeDtypeStruct((1, H), jnp.float32),
            jax.ShapeDtypeStruct((1, H), jnp.float32),
        ],
    )(ga0, ga1, ga2, w_flat, ff_flat, W1, b1.reshape(1, H))

    mean1 = s1 / cnt
    var1 = ss1 / cnt - mean1 * mean1
    sc1 = g1.reshape(1, H) / jnp.sqrt(var1 + eps)
    sh1 = be1.reshape(1, H) - mean1 * sc1

    y2, s2, ss2 = pl.pallas_call(
        _mlp2_kernel,
        grid=(P // blk2,),
        in_specs=[
            pl.BlockSpec((blk2, H), lambda i: (i, 0)),
            pl.BlockSpec((1, H), lambda i: (0, 0)),
            pl.BlockSpec((1, H), lambda i: (0, 0)),
            pl.BlockSpec((H, H), lambda i: (0, 0)),
            pl.BlockSpec((1, H), lambda i: (0, 0)),
        ],
        out_specs=[
            pl.BlockSpec((blk2, H), lambda i: (i, 0)),
            pl.BlockSpec((1, H), lambda i: (0, 0)),
            pl.BlockSpec((1, H), lambda i: (0, 0)),
        ],
        out_shape=[
            jax.ShapeDtypeStruct((P, H), jnp.float32),
            jax.ShapeDtypeStruct((1, H), jnp.float32),
            jax.ShapeDtypeStruct((1, H), jnp.float32),
        ],
    )(y1, sc1, sh1, W2, b2.reshape(1, H))

    mean2 = s2 / cnt
    var2 = ss2 / cnt - mean2 * mean2
    sc2 = g2.reshape(1, H) / jnp.sqrt(var2 + eps)
    sh2 = be2.reshape(1, H) - mean2 * sc2

    out = pl.pallas_call(
        _bn_out_kernel,
        grid=(P // blk2,),
        in_specs=[
            pl.BlockSpec((blk2, H), lambda i: (i, 0)),
            pl.BlockSpec((1, H), lambda i: (0, 0)),
            pl.BlockSpec((1, H), lambda i: (0, 0)),
        ],
        out_specs=pl.BlockSpec((blk2, H), lambda i: (i, 0)),
        out_shape=jax.ShapeDtypeStruct((P, H), jnp.float32),
    )(y2, sc2, sh2)

    return out.reshape(B, N, H)


# speedup vs baseline: 43.2530x; 1.0016x over previous
import functools

import jax
import jax.numpy as jnp
from jax import lax
from jax.experimental import pallas as pl
from jax.experimental.pallas import tpu as pltpu
from jax.experimental.pallas import tpu_sc as plsc

_HI = jax.lax.Precision.HIGHEST

_NC = 2
_NS = 16
_NW = _NC * _NS


def _knn_kernel(xyzf_ref, xyzct_ref, idx_ref, w_ref):
    b = pl.program_id(0)

    x = xyzf_ref[0]
    ct = xyzct_ref[0]
    d = -2.0 * jax.lax.dot_general(x, ct, (((1,), (0,)), ((), ())))
    d = d + jnp.sum(x * x, axis=1, keepdims=True)
    d = d + jnp.sum(ct * ct, axis=0, keepdims=True)

    m = d.shape[1]
    coli = jax.lax.broadcasted_iota(jnp.int32, d.shape, 1)
    dc = jnp.maximum(d, jnp.float32(1e-10))
    key = jax.lax.bitcast_convert_type(dc, jnp.int32)
    key = jnp.bitwise_or(jnp.bitwise_and(key, jnp.int32(-2048)), coli)
    kw = jax.lax.bitcast_convert_type(key, jnp.float32)

    mks = []
    for k in range(3):
        mk = jnp.min(kw, axis=1, keepdims=True)
        mks.append(mk)
        if k < 2:
            kw = jnp.where(kw == mk, jnp.float32(jnp.inf), kw)

    k3 = jax.lax.bitcast_convert_type(jnp.concatenate(mks, axis=1),
                                      jnp.int32)
    ig = jnp.bitwise_and(k3, jnp.int32(2047)) + b * m
    d3 = jax.lax.bitcast_convert_type(jnp.bitwise_or(
        jnp.bitwise_and(k3, jnp.int32(-2048)), jnp.int32(1024)), jnp.float32)
    d3 = jnp.maximum(d3, jnp.float32(1e-10))
    w = 1.0 / d3
    w = w / jnp.sum(w, axis=1, keepdims=True)

    idx_ref[...] = jnp.transpose(ig, (1, 0)).reshape(3, 1, -1)
    w_ref[...] = jnp.transpose(w, (1, 0)).reshape(3, 1, -1)


def _sc_gather_body(idx_hbm, w_hbm, fc_hbm, out_hbm,
                    i0, i1, i2, w0v, w1v, w2v, r0, r1, r2, sem):
    p_per_w = idx_hbm.shape[2] // _NW
    chunk = i0.shape[0]
    wid = lax.axis_index("s") * _NC + lax.axis_index("c")
    base = wid * p_per_w

    def chunk_body(ci, carry):
        cbase = base + ci * chunk
        pltpu.sync_copy(idx_hbm.at[0, 0, pl.ds(cbase, chunk)], i0)
        pltpu.sync_copy(idx_hbm.at[1, 0, pl.ds(cbase, chunk)], i1)
        pltpu.sync_copy(idx_hbm.at[2, 0, pl.ds(cbase, chunk)], i2)
        pltpu.sync_copy(w_hbm.at[0, 0, pl.ds(cbase, chunk)], w0v)
        pltpu.sync_copy(w_hbm.at[1, 0, pl.ds(cbase, chunk)], w1v)
        pltpu.sync_copy(w_hbm.at[2, 0, pl.ds(cbase, chunk)], w2v)
        c0 = pltpu.async_copy(fc_hbm.at[i0], r0, sem)
        c1 = pltpu.async_copy(fc_hbm.at[i1], r1, sem)
        c2 = pltpu.async_copy(fc_hbm.at[i2], r2, sem)
        c0.wait()
        c1.wait()
        c2.wait()

        def g_body(g, carry2):
            gb = g * 16
            wr0 = w0v[pl.ds(gb, 16)]
            wr1 = w1v[pl.ds(gb, 16)]
            wr2 = w2v[pl.ds(gb, 16)]
            for j in range(16):
                jv = jnp.full((16,), j, jnp.int32)
                w0 = jnp.take(wr0, jv)
                w1 = jnp.take(wr1, jv)
                w2 = jnp.take(wr2, jv)
                pp = gb + j
                for c in range(4):
                    s = pl.ds(c * 16, 16)
                    r0[pp, s] = (w0 * r0[pp, s] + w1 * r1[pp, s]
                                 + w2 * r2[pp, s])
            return carry2

        lax.fori_loop(0, chunk // 16, g_body, 0)
        pltpu.sync_copy(r0, out_hbm.at[pl.ds(cbase, chunk)])
        return carry

    lax.fori_loop(0, p_per_w // chunk, chunk_body, 0)


def _mlp1_kernel(it_ref, ff_ref, w1_ref, b1_ref, y1_ref, s_ref, ss_ref):
    i = pl.program_id(0)
    ff = ff_ref[...]
    w1 = w1_ref[...]
    c2 = w1.shape[1] - ff.shape[1]
    it = it_ref[:, :c2]
    y = jax.lax.dot_general(it, w1[:, :c2], (((1,), (1,)), ((), ())),
                            precision=_HI)
    y = y + jax.lax.dot_general(ff, w1[:, c2:], (((1,), (1,)), ((), ())),
                                precision=_HI)
    y = y + b1_ref[...]
    y1_ref[...] = y

    @pl.when(i == 0)
    def _init():
        s_ref[...] = jnp.zeros_like(s_ref)
        ss_ref[...] = jnp.zeros_like(ss_ref)

    s_ref[...] += jnp.sum(y, axis=0, keepdims=True)
    ss_ref[...] += jnp.sum(y * y, axis=0, keepdims=True)


def _mlp2_kernel(y_ref, sc_ref, sh_ref, w2_ref, b2_ref,
                 o_ref, s_ref, ss_ref):
    i = pl.program_id(0)
    z = jnp.maximum(y_ref[...] * sc_ref[...] + sh_ref[...], 0.0)
    y = jax.lax.dot_general(z, w2_ref[...], (((1,), (1,)), ((), ())),
                            precision=_HI) + b2_ref[...]
    o_ref[...] = y

    @pl.when(i == 0)
    def _init():
        s_ref[...] = jnp.zeros_like(s_ref)
        ss_ref[...] = jnp.zeros_like(ss_ref)

    s_ref[...] += jnp.sum(y, axis=0, keepdims=True)
    ss_ref[...] += jnp.sum(y * y, axis=0, keepdims=True)


def _bn_out_kernel(y_ref, sc_ref, sh_ref, o_ref):
    o_ref[...] = jnp.maximum(y_ref[...] * sc_ref[...] + sh_ref[...], 0.0)


def kernel(xyz_fine, xyz_coarse, feats_fine, feats_coarse,
           W1, b1, g1, be1, W2, b2, g2, be2):
    B, N, _ = xyz_fine.shape
    M = xyz_coarse.shape[1]
    C1 = feats_fine.shape[2]
    C2 = feats_coarse.shape[2]
    H = W1.shape[0]
    P = B * N
    eps = jnp.float32(1e-5)
    cnt = jnp.float32(P)

    blk = 512
    nb = N // blk
    xyz_coarse_t = jnp.transpose(xyz_coarse, (0, 2, 1))

    idx_t, w_t = pl.pallas_call(
        _knn_kernel,
        grid=(B, nb),
        in_specs=[
            pl.BlockSpec((1, blk, 3), lambda b, n: (b, n, 0)),
            pl.BlockSpec((1, 3, M), lambda b, n: (b, 0, 0)),
        ],
        out_specs=[
            pl.BlockSpec((3, 1, blk), lambda b, n: (0, 0, b * (N // 512) + n)),
            pl.BlockSpec((3, 1, blk), lambda b, n: (0, 0, b * (N // 512) + n)),
        ],
        out_shape=[
            jax.ShapeDtypeStruct((3, 1, P), jnp.int32),
            jax.ShapeDtypeStruct((3, 1, P), jnp.float32),
        ],
    )(xyz_fine, xyz_coarse_t)

    fc_flat = feats_coarse.reshape(B * M, C2)
    fc_pad = jnp.pad(fc_flat, ((0, 0), (0, 128 - C2)))
    chunk = 256
    sc_gather = functools.partial(
        pl.kernel,
        mesh=plsc.VectorSubcoreMesh(core_axis_name="c", subcore_axis_name="s"),
        out_type=jax.ShapeDtypeStruct((P, 128), jnp.float32),
        scratch_types=[
            pltpu.VMEM((chunk,), jnp.int32),
            pltpu.VMEM((chunk,), jnp.int32),
            pltpu.VMEM((chunk,), jnp.int32),
            pltpu.VMEM((chunk,), jnp.float32),
            pltpu.VMEM((chunk,), jnp.float32),
            pltpu.VMEM((chunk,), jnp.float32),
            pltpu.VMEM((chunk, 128), jnp.float32),
            pltpu.VMEM((chunk, 128), jnp.float32),
            pltpu.VMEM((chunk, 128), jnp.float32),
            pltpu.SemaphoreType.DMA,
        ],
    )(_sc_gather_body)
    interp = sc_gather(idx_t, w_t, fc_pad)

    ff_flat = feats_fine.reshape(P, C1)
    blk2 = 2048
    y1, s1, ss1 = pl.pallas_call(
        _mlp1_kernel,
        grid=(P // blk2,),
        in_specs=[
            pl.BlockSpec((blk2, 128), lambda i: (i, 0)),
            pl.BlockSpec((blk2, C1), lambda i: (i, 0)),
            pl.BlockSpec((H, C1 + C2), lambda i: (0, 0)),
            pl.BlockSpec((1, H), lambda i: (0, 0)),
        ],
        out_specs=[
            pl.BlockSpec((blk2, H), lambda i: (i, 0)),
            pl.BlockSpec((1, H), lambda i: (0, 0)),
            pl.BlockSpec((1, H), lambda i: (0, 0)),
        ],
        out_shape=[
            jax.ShapeDtypeStruct((P, H), jnp.float32),
            jax.ShapeDtypeStruct((1, H), jnp.float32),
            jax.ShapeDtypeStruct((1, H), jnp.float32),
        ],
    )(interp, ff_flat, W1, b1.reshape(1, H))

    mean1 = s1 / cnt
    var1 = ss1 / cnt - mean1 * mean1
    sc1 = g1.reshape(1, H) / jnp.sqrt(var1 + eps)
    sh1 = be1.reshape(1, H) - mean1 * sc1

    y2, s2, ss2 = pl.pallas_call(
        _mlp2_kernel,
        grid=(P // blk2,),
        in_specs=[
            pl.BlockSpec((blk2, H), lambda i: (i, 0)),
            pl.BlockSpec((1, H), lambda i: (0, 0)),
            pl.BlockSpec((1, H), lambda i: (0, 0)),
            pl.BlockSpec((H, H), lambda i: (0, 0)),
            pl.BlockSpec((1, H), lambda i: (0, 0)),
        ],
        out_specs=[
            pl.BlockSpec((blk2, H), lambda i: (i, 0)),
            pl.BlockSpec((1, H), lambda i: (0, 0)),
            pl.BlockSpec((1, H), lambda i: (0, 0)),
        ],
        out_shape=[
            jax.ShapeDtypeStruct((P, H), jnp.float32),
            jax.ShapeDtypeStruct((1, H), jnp.float32),
            jax.ShapeDtypeStruct((1, H), jnp.float32),
        ],
    )(y1, sc1, sh1, W2, b2.reshape(1, H))

    mean2 = s2 / cnt
    var2 = ss2 / cnt - mean2 * mean2
    sc2 = g2.reshape(1, H) / jnp.sqrt(var2 + eps)
    sh2 = be2.reshape(1, H) - mean2 * sc2

    out = pl.pallas_call(
        _bn_out_kernel,
        grid=(P // blk2,),
        in_specs=[
            pl.BlockSpec((blk2, H), lambda i: (i, 0)),
            pl.BlockSpec((1, H), lambda i: (0, 0)),
            pl.BlockSpec((1, H), lambda i: (0, 0)),
        ],
        out_specs=pl.BlockSpec((blk2, H), lambda i: (i, 0)),
        out_shape=jax.ShapeDtypeStruct((P, H), jnp.float32),
    )(y2, sc2, sh2)

    return out.reshape(B, N, H)
